# baseline (speedup 1.0000x reference)
import functools

import jax
import jax.numpy as jnp
from jax import lax
from jax.experimental import pallas as pl
from jax.experimental.pallas import tpu as pltpu
from jax.experimental.pallas import tpu_sc as plsc

BATCH = 8192
D = 2048
L = 16
NC = 2
NS = 16
NW = NC * NS
ROWS_PER_W = BATCH // NW
CHUNK = 8
N_CHUNKS = ROWS_PER_W // CHUNK
JBLKS = D // L
NBI = 4
NBO = 2


def _gather_chunk(in_chunk, out_chunk, sel_v, row_splats):
  @plsc.parallel_loop(0, JBLKS, unroll=8)
  def _(j):
    cols = sel_v[pl.ds(j * L, L)]
    off = j * L
    for r in range(CHUNK):
      vals = plsc.load_gather(in_chunk, [row_splats[r], cols])
      out_chunk[r, pl.ds(off, L)] = vals


def _body(tgt_hbm, perm_hbm, inv_hbm, flag_hbm, out_hbm,
          perm_v, inv_v, flag_v, sel_v, in_bufs, out_bufs, sems):
  wid = lax.axis_index("s") * NC + lax.axis_index("c")
  row0 = wid * ROWS_PER_W

  def chunk_slice(c):
    return pl.ds(row0 + c * CHUNK, CHUNK)

  h_in = [None] * NBI
  for p in range(NBI - 1):
    h_in[p] = pltpu.async_copy(tgt_hbm.at[chunk_slice(p)], in_bufs[p],
                               sems.at[p])

  hp = pltpu.async_copy(perm_hbm, perm_v, sems.at[NBI + NBO])
  hq = pltpu.async_copy(inv_hbm, inv_v, sems.at[NBI + NBO + 1])
  hf = pltpu.async_copy(flag_hbm, flag_v, sems.at[NBI + NBO + 2])
  hp.wait()
  hq.wait()
  hf.wait()
  use_inv = flag_v[...] != 0

  def sel_body(j, carry):
    p = perm_v[pl.ds(j * L, L)]
    q = inv_v[pl.ds(j * L, L)]
    sel_v[pl.ds(j * L, L)] = lax.select(use_inv, q, p)
    return carry

  lax.fori_loop(0, JBLKS, sel_body, 0)

  row_splats = [jnp.full((L,), r, jnp.int32) for r in range(CHUNK)]
  h_out = [None] * NBO
  for c in range(N_CHUNKS):
    bi = c % NBI
    bo = c % NBO
    ni = (c + NBI - 1) % NBI
    if c + NBI - 1 < N_CHUNKS:
      h_in[ni] = pltpu.async_copy(tgt_hbm.at[chunk_slice(c + NBI - 1)],
                                  in_bufs[ni], sems.at[ni])
    if c >= NBO:
      h_out[bo].wait()
    h_in[bi].wait()
    _gather_chunk(in_bufs[bi], out_bufs[bo], sel_v, row_splats)
    h_out[bo] = pltpu.async_copy(out_bufs[bo], out_hbm.at[chunk_slice(c)],
                                 sems.at[NBI + bo])
  for bo in range(NBO):
    h_out[bo].wait()


@functools.partial(
    pl.kernel,
    mesh=plsc.VectorSubcoreMesh(core_axis_name="c", subcore_axis_name="s"),
    out_type=jax.ShapeDtypeStruct((BATCH, D), jnp.float32),
    compiler_params=pltpu.CompilerParams(
        needs_layout_passes=False, use_tc_tiling_on_sc=True),
    scratch_types=[
        pltpu.VMEM((D,), jnp.int32),
        pltpu.VMEM((D,), jnp.int32),
        pltpu.VMEM((L,), jnp.int32),
        pltpu.VMEM((D,), jnp.int32),
    ] + [pltpu.VMEM((CHUNK, D), jnp.float32) for _ in range(NBI + NBO)] + [
        pltpu.SemaphoreType.DMA((NBI + NBO + 3,)),
    ],
)
def _permute_sc(tgt_hbm, perm_hbm, inv_hbm, flag_hbm, out_hbm,
                perm_v, inv_v, flag_v, sel_v, *bufs_and_sems):
  bufs = bufs_and_sems[:NBI + NBO]
  sems = bufs_and_sems[NBI + NBO]
  _body(tgt_hbm, perm_hbm, inv_hbm, flag_hbm, out_hbm,
        perm_v, inv_v, flag_v, sel_v, bufs[:NBI], bufs[NBI:], sems)


@jax.jit
def kernel(target, permutation, inv_permutation, inverse):
  flag = jnp.broadcast_to(jnp.asarray(inverse, jnp.int32), (L,))
  return _permute_sc(target, permutation, inv_permutation, flag)


# speedup vs baseline: 1.0035x; 1.0035x over previous
import functools

import jax
import jax.numpy as jnp
from jax import lax
from jax.experimental import pallas as pl
from jax.experimental.pallas import tpu as pltpu
from jax.experimental.pallas import tpu_sc as plsc

BATCH = 8192
D = 2048
L = 16
NC = 2
NS = 16
NW = NC * NS
ROWS_PER_W = BATCH // NW
CHUNK = 8
SUPER = 16
N_SUPER = ROWS_PER_W // SUPER
JBLKS = D // L
NBO = 3


def _gather_chunk(in_chunk, out_chunk, sel_v, row_splats):
  @plsc.parallel_loop(0, JBLKS, unroll=8)
  def _(j):
    cols = sel_v[pl.ds(j * L, L)]
    off = j * L
    for r in range(CHUNK):
      vals = plsc.load_gather(in_chunk, [row_splats[r], cols])
      out_chunk[r, pl.ds(off, L)] = vals


def _body(tgt_hbm, perm_hbm, inv_hbm, flag_hbm, out_hbm,
          perm_v, inv_v, flag_v, sel_v, in_bufs, out_bufs, sems):
  wid = lax.axis_index("s") * NC + lax.axis_index("c")
  row0 = wid * ROWS_PER_W

  def super_slice(s):
    return pl.ds(row0 + s * SUPER, SUPER)

  def chunk_slice(c):
    return pl.ds(row0 + c * CHUNK, CHUNK)

  h_in = [None, None]
  h_in[0] = pltpu.async_copy(tgt_hbm.at[super_slice(0)], in_bufs[0],
                             sems.at[0])

  hp = pltpu.async_copy(perm_hbm, perm_v, sems.at[2 + NBO])
  hq = pltpu.async_copy(inv_hbm, inv_v, sems.at[2 + NBO + 1])
  hf = pltpu.async_copy(flag_hbm, flag_v, sems.at[2 + NBO + 2])
  hp.wait()
  hq.wait()
  hf.wait()
  use_inv = flag_v[...] != 0

  def sel_body(j, carry):
    p = perm_v[pl.ds(j * L, L)]
    q = inv_v[pl.ds(j * L, L)]
    sel_v[pl.ds(j * L, L)] = lax.select(use_inv, q, p)
    return carry

  lax.fori_loop(0, JBLKS, sel_body, 0)

  row_splats = [jnp.full((L,), r, jnp.int32) for r in range(SUPER)]
  h_out = [None] * NBO
  for s in range(N_SUPER):
    bi = s % 2
    if s + 1 < N_SUPER:
      h_in[(s + 1) % 2] = pltpu.async_copy(tgt_hbm.at[super_slice(s + 1)],
                                           in_bufs[(s + 1) % 2],
                                           sems.at[(s + 1) % 2])
    h_in[bi].wait()
    for h in range(2):
      c = 2 * s + h
      bo = c % NBO
      if c >= NBO:
        h_out[bo].wait()
      _gather_chunk(in_bufs[bi], out_bufs[bo], sel_v,
                    row_splats[h * CHUNK:(h + 1) * CHUNK])
      h_out[bo] = pltpu.async_copy(out_bufs[bo], out_hbm.at[chunk_slice(c)],
                                   sems.at[2 + bo])
  for bo in range(NBO):
    h_out[bo].wait()


@functools.partial(
    pl.kernel,
    mesh=plsc.VectorSubcoreMesh(core_axis_name="c", subcore_axis_name="s"),
    out_type=jax.ShapeDtypeStruct((BATCH, D), jnp.float32),
    compiler_params=pltpu.CompilerParams(
        needs_layout_passes=False, use_tc_tiling_on_sc=True),
    scratch_types=[
        pltpu.VMEM((D,), jnp.int32),
        pltpu.VMEM((D,), jnp.int32),
        pltpu.VMEM((L,), jnp.int32),
        pltpu.VMEM((D,), jnp.int32),
    ] + [pltpu.VMEM((SUPER, D), jnp.float32) for _ in range(2)]
      + [pltpu.VMEM((CHUNK, D), jnp.float32) for _ in range(NBO)] + [
        pltpu.SemaphoreType.DMA((2 + NBO + 3,)),
    ],
)
def _permute_sc(tgt_hbm, perm_hbm, inv_hbm, flag_hbm, out_hbm,
                perm_v, inv_v, flag_v, sel_v, *bufs_and_sems):
  bufs = bufs_and_sems[:2 + NBO]
  sems = bufs_and_sems[2 + NBO]
  _body(tgt_hbm, perm_hbm, inv_hbm, flag_hbm, out_hbm,
        perm_v, inv_v, flag_v, sel_v, bufs[:2], bufs[2:], sems)


@jax.jit
def kernel(target, permutation, inv_permutation, inverse):
  flag = jnp.broadcast_to(jnp.asarray(inverse, jnp.int32), (L,))
  return _permute_sc(target, permutation, inv_permutation, flag)


# speedup vs baseline: 1.0109x; 1.0073x over previous
import functools

import jax
import jax.numpy as jnp
from jax import lax
from jax.experimental import pallas as pl
from jax.experimental.pallas import tpu as pltpu
from jax.experimental.pallas import tpu_sc as plsc

BATCH = 8192
D = 2048
L = 16
NC = 2
NS = 16
NW = NC * NS
ROWS_PER_W = BATCH // NW
CHUNK = 8
N_CHUNKS = ROWS_PER_W // CHUNK
JBLKS = D // L
NBUF = 3


def _gather_chunk(in_chunk, out_chunk, sel_v, row_splats):
  @plsc.parallel_loop(0, JBLKS, unroll=8)
  def _(j):
    cols = sel_v[pl.ds(j * L, L)]
    off = j * L
    for r in range(CHUNK):
      vals = plsc.load_gather(in_chunk, [row_splats[r], cols])
      out_chunk[r, pl.ds(off, L)] = vals


def _body(tgt_hbm, perm_hbm, inv_hbm, flag_hbm, out_hbm,
          perm_v, inv_v, flag_v, sel_v, in_bufs, out_bufs, sems):
  wid = lax.axis_index("s") * NC + lax.axis_index("c")
  row0 = wid * ROWS_PER_W

  def chunk_slice(c):
    return pl.ds(row0 + c * CHUNK, CHUNK)

  h_in = [None] * NBUF
  for p in range(NBUF - 1):
    h_in[p] = pltpu.async_copy(tgt_hbm.at[chunk_slice(p)], in_bufs[p],
                               sems.at[p])

  hp = pltpu.async_copy(perm_hbm, perm_v, sems.at[2 * NBUF])
  hq = pltpu.async_copy(inv_hbm, inv_v, sems.at[2 * NBUF + 1])
  hf = pltpu.async_copy(flag_hbm, flag_v, sems.at[2 * NBUF + 2])
  hp.wait()
  hq.wait()
  hf.wait()
  use_inv = flag_v[...] != 0

  def sel_body(j, carry):
    p = perm_v[pl.ds(j * L, L)]
    q = inv_v[pl.ds(j * L, L)]
    sel_v[pl.ds(j * L, L)] = lax.select(use_inv, q, p)
    return carry

  lax.fori_loop(0, JBLKS, sel_body, 0)

  row_splats = [jnp.full((L,), r, jnp.int32) for r in range(CHUNK)]
  h_out = [None] * NBUF
  for c in range(N_CHUNKS):
    b = c % NBUF
    nb = (c + NBUF - 1) % NBUF
    if c + NBUF - 1 < N_CHUNKS:
      h_in[nb] = pltpu.async_copy(tgt_hbm.at[chunk_slice(c + NBUF - 1)],
                                  in_bufs[nb], sems.at[nb])
    if c >= NBUF:
      h_out[b].wait()
    h_in[b].wait()
    _gather_chunk(in_bufs[b], out_bufs[b], sel_v, row_splats)
    h_out[b] = pltpu.async_copy(out_bufs[b], out_hbm.at[chunk_slice(c)],
                                sems.at[NBUF + b])
  for b in range(NBUF):
    h_out[b].wait()


@functools.partial(
    pl.kernel,
    mesh=plsc.VectorSubcoreMesh(core_axis_name="c", subcore_axis_name="s"),
    out_type=jax.ShapeDtypeStruct((BATCH, D), jnp.float32),
    compiler_params=pltpu.CompilerParams(
        needs_layout_passes=False, use_tc_tiling_on_sc=True),
    scratch_types=[
        pltpu.VMEM((D,), jnp.int32),
        pltpu.VMEM((D,), jnp.int32),
        pltpu.VMEM((L,), jnp.int32),
        pltpu.VMEM((D,), jnp.int32),
    ] + [pltpu.VMEM((CHUNK, D), jnp.float32) for _ in range(2 * NBUF)] + [
        pltpu.SemaphoreType.DMA((2 * NBUF + 3,)),
    ],
)
def _permute_sc(tgt_hbm, perm_hbm, inv_hbm, flag_hbm, out_hbm,
                perm_v, inv_v, flag_v, sel_v, *bufs_and_sems):
  bufs = bufs_and_sems[:2 * NBUF]
  sems = bufs_and_sems[2 * NBUF]
  _body(tgt_hbm, perm_hbm, inv_hbm, flag_hbm, out_hbm,
        perm_v, inv_v, flag_v, sel_v, bufs[:NBUF], bufs[NBUF:], sems)


@jax.jit
def kernel(target, permutation, inv_permutation, inverse):
  flag = jnp.broadcast_to(jnp.asarray(inverse, jnp.int32), (L,))
  return _permute_sc(target, permutation, inv_permutation, flag)
